# Initial kernel scaffold; baseline (speedup 1.0000x reference)
import jax
import jax.numpy as jnp
from jax.experimental import pallas as pl


def kernel(z, edge_index, node_id, W1, b1, W2, b2):
    raise NotImplementedError("write your pallas kernel here")


# speedup vs baseline: 5.6873x; 5.6873x over previous
import functools

import jax
import jax.numpy as jnp
from jax import lax
from jax.experimental import pallas as pl
from jax.experimental.pallas import tpu as pltpu
from jax.experimental.pallas import tpu_sc as plsc

NC = 2
NS = 16
NW = NC * NS
L = 16


def _tables_body(z_ref, w1_ref, b1_ref, xn_ref, p_ref, q_ref):
    c = jnp.dot(xn_ref[...], w1_ref[256:384, :],
                preferred_element_type=jnp.float32) + b1_ref[...]
    c = c * 0.5
    p_ref[...] = jnp.dot(z_ref[...], w1_ref[0:128, :],
                         preferred_element_type=jnp.float32) + c
    q_ref[...] = jnp.dot(z_ref[...], w1_ref[128:256, :],
                         preferred_element_type=jnp.float32) + c


def _build_tables(z, W1, b1, xn):
    n, _ = z.shape
    h = W1.shape[1]
    out = jax.ShapeDtypeStruct((n, h), jnp.float32)
    return pl.pallas_call(_tables_body, out_shape=(out, out))(z, W1, b1, xn)


def _sc_edge_body(p_hbm, q_hbm, rows_hbm, cols_hbm, w2_hbm, out_hbm,
                  ridx_v, cidx_v, qrows_v, prows_v, out_v, w2_v, sem1, sem2,
                  *, edges_per_w, blk, hidden):
    wid = lax.axis_index("c") * NS + lax.axis_index("s")
    base = wid * edges_per_w
    pltpu.sync_copy(w2_hbm, w2_v)
    nchunk = hidden // L

    @pl.loop(0, edges_per_w // blk)
    def _step(step):
        off = base + step * blk
        pltpu.sync_copy(rows_hbm.at[pl.ds(off, blk)], ridx_v)
        pltpu.sync_copy(cols_hbm.at[pl.ds(off, blk)], cidx_v)
        cp_q = pltpu.async_copy(q_hbm.at[ridx_v], qrows_v, sem1)
        cp_p = pltpu.async_copy(p_hbm.at[cidx_v], prows_v, sem2)
        cp_q.wait()
        cp_p.wait()

        @pl.loop(0, blk)
        def _edge(e):
            s = None
            for ch in range(nchunk):
                sl = pl.ds(ch * L, L)
                t = jnp.maximum(prows_v[e, sl] + qrows_v[e, sl], 0.0) * w2_v[sl]
                s = t if s is None else s + t
            out_v[e, :] = s

        pltpu.sync_copy(out_v, out_hbm.at[pl.ds(off, blk), :])


def _sc_edge(P, Q, rows, cols, w2):
    n, hidden = P.shape
    e = rows.shape[0]
    edges_per_w = e // NW
    blk = 400
    mesh = plsc.VectorSubcoreMesh(core_axis_name="c", subcore_axis_name="s")
    body = functools.partial(_sc_edge_body, edges_per_w=edges_per_w, blk=blk,
                             hidden=hidden)
    run = pl.kernel(
        body,
        out_type=jax.ShapeDtypeStruct((e, L), jnp.float32),
        mesh=mesh,
        compiler_params=pltpu.CompilerParams(use_tc_tiling_on_sc=False),
        scratch_types=[
            pltpu.VMEM((blk,), jnp.int32),
            pltpu.VMEM((blk,), jnp.int32),
            pltpu.VMEM((blk, hidden), jnp.float32),
            pltpu.VMEM((blk, hidden), jnp.float32),
            pltpu.VMEM((blk, L), jnp.float32),
            pltpu.VMEM((hidden,), jnp.float32),
            pltpu.SemaphoreType.DMA,
            pltpu.SemaphoreType.DMA,
        ],
    )
    return run(P, Q, rows, cols, w2)


def _reduce_body(x_ref, b2_ref, o_ref):
    sel = (lax.broadcasted_iota(jnp.int32, (128, 8), 0) // L
           == lax.broadcasted_iota(jnp.int32, (128, 8), 1))
    ones = sel.astype(jnp.float32)
    o_ref[...] = jnp.dot(x_ref[...], ones,
                         preferred_element_type=jnp.float32) + b2_ref[0, 0]


def _reduce(x2, b2):
    rows = x2.shape[0]
    grid = 10
    blk = rows // grid
    return pl.pallas_call(
        _reduce_body,
        grid=(grid,),
        in_specs=[pl.BlockSpec((blk, 128), lambda i: (i, 0)),
                  pl.BlockSpec((1, 1), lambda i: (0, 0))],
        out_specs=pl.BlockSpec((blk, 8), lambda i: (i, 0)),
        out_shape=jax.ShapeDtypeStruct((rows, 8), jnp.float32),
    )(x2, b2)


def kernel(z, edge_index, node_id, W1, b1, W2, b2):
    e = edge_index.shape[1]
    rows = edge_index[0]
    cols = edge_index[1]
    xn = lax.dynamic_slice_in_dim(z, node_id, 1, axis=0)
    P, Q = _build_tables(z, W1, b1.reshape(1, -1), xn)
    out16 = _sc_edge(P, Q, rows, cols, W2[:, 0])
    out8 = _reduce(out16.reshape(e * L // 128, 128), b2.reshape(1, 1))
    return out8.reshape(e, 1)


# speedup vs baseline: 11.3459x; 1.9950x over previous
import functools

import jax
import jax.numpy as jnp
from jax import lax
from jax.experimental import pallas as pl
from jax.experimental.pallas import tpu as pltpu
from jax.experimental.pallas import tpu_sc as plsc

NC = 2
NS = 16
NW = NC * NS
L = 16


def _tables_body(z_ref, w1_ref, b1_ref, xn_ref, p_ref, q_ref):
    c = jnp.dot(xn_ref[...], w1_ref[256:384, :],
                preferred_element_type=jnp.float32) + b1_ref[...]
    c = c * 0.5
    p_ref[...] = jnp.dot(z_ref[...], w1_ref[0:128, :],
                         preferred_element_type=jnp.float32) + c
    q_ref[...] = jnp.dot(z_ref[...], w1_ref[128:256, :],
                         preferred_element_type=jnp.float32) + c


def _build_tables(z, W1, b1, xn):
    n, _ = z.shape
    h = W1.shape[1]
    out = jax.ShapeDtypeStruct((n, h), jnp.float32)
    return pl.pallas_call(_tables_body, out_shape=(out, out))(z, W1, b1, xn)


def _sc_edge_body(p_hbm, q_hbm, rows_hbm, cols_hbm, w2_hbm, out_hbm,
                  ridx_v, cidx_v, qrows0, qrows1, prows0, prows1,
                  out0, out1, w2_v, qs0, qs1, ps0, ps1, os0, os1,
                  *, edges_per_w, blk, hidden):
    wid = lax.axis_index("c") * NS + lax.axis_index("s")
    base = wid * edges_per_w
    nstep = edges_per_w // blk
    nchunk = hidden // L
    pltpu.sync_copy(w2_hbm, w2_v)
    pltpu.sync_copy(rows_hbm.at[pl.ds(base, edges_per_w)], ridx_v)
    pltpu.sync_copy(cols_hbm.at[pl.ds(base, edges_per_w)], cidx_v)

    qrows = (qrows0, qrows1)
    prows = (prows0, prows1)
    outs = (out0, out1)
    qsems = (qs0, qs1)
    psems = (ps0, ps1)
    osems = (os0, os1)

    def gather_descs(i, b):
        off = i * blk
        dq = pltpu.make_async_copy(
            q_hbm.at[ridx_v.at[pl.ds(off, blk)]], qrows[b], qsems[b])
        dp = pltpu.make_async_copy(
            p_hbm.at[cidx_v.at[pl.ds(off, blk)]], prows[b], psems[b])
        return dq, dp

    def out_desc(i, b):
        return pltpu.make_async_copy(
            outs[b], out_hbm.at[pl.ds(base + i * blk, blk), :], osems[b])

    dq, dp = gather_descs(0, 0)
    dq.start()
    dp.start()

    def compute(b):
        pr, qr, ob = prows[b], qrows[b], outs[b]

        @functools.partial(plsc.parallel_loop, 0, blk, unroll=4)
        def _edge(e):
            s = None
            for ch in range(nchunk):
                sl = pl.ds(ch * L, L)
                t = jnp.maximum(pr[e, sl] + qr[e, sl], 0.0) * w2_v[sl]
                s = t if s is None else s + t
            ob[e, :] = s

    def process(j, b):
        @pl.when(j + 1 < nstep)
        def _():
            nq, np_ = gather_descs(j + 1, 1 - b)
            nq.start()
            np_.start()

        wq, wp = gather_descs(j, b)
        wq.wait()
        wp.wait()

        @pl.when(j >= 2)
        def _():
            out_desc(j - 2, b).wait()

        compute(b)
        out_desc(j, b).start()

    @pl.loop(0, nstep, step=2)
    def _(j):
        process(j, 0)
        process(j + 1, 1)

    out_desc(nstep - 2, 0).wait()
    out_desc(nstep - 1, 1).wait()


def _sc_edge(P, Q, rows, cols, w2):
    n, hidden = P.shape
    e = rows.shape[0]
    edges_per_w = e // NW
    blk = 200
    mesh = plsc.VectorSubcoreMesh(core_axis_name="c", subcore_axis_name="s")
    body = functools.partial(_sc_edge_body, edges_per_w=edges_per_w, blk=blk,
                             hidden=hidden)
    run = pl.kernel(
        body,
        out_type=jax.ShapeDtypeStruct((e, L), jnp.float32),
        mesh=mesh,
        compiler_params=pltpu.CompilerParams(use_tc_tiling_on_sc=False),
        scratch_types=[
            pltpu.VMEM((edges_per_w,), jnp.int32),
            pltpu.VMEM((edges_per_w,), jnp.int32),
            pltpu.VMEM((blk, hidden), jnp.float32),
            pltpu.VMEM((blk, hidden), jnp.float32),
            pltpu.VMEM((blk, hidden), jnp.float32),
            pltpu.VMEM((blk, hidden), jnp.float32),
            pltpu.VMEM((blk, L), jnp.float32),
            pltpu.VMEM((blk, L), jnp.float32),
            pltpu.VMEM((hidden,), jnp.float32),
            pltpu.SemaphoreType.DMA,
            pltpu.SemaphoreType.DMA,
            pltpu.SemaphoreType.DMA,
            pltpu.SemaphoreType.DMA,
            pltpu.SemaphoreType.DMA,
            pltpu.SemaphoreType.DMA,
        ],
    )
    return run(P, Q, rows, cols, w2)


def _reduce_body(x_ref, b2_ref, o_ref):
    sel = (lax.broadcasted_iota(jnp.int32, (128, 8), 0) // L
           == lax.broadcasted_iota(jnp.int32, (128, 8), 1))
    ones = sel.astype(jnp.float32)
    o_ref[...] = jnp.dot(x_ref[...], ones,
                         preferred_element_type=jnp.float32) + b2_ref[0, 0]


def _reduce(x2, b2):
    rows = x2.shape[0]
    grid = 10
    blk = rows // grid
    return pl.pallas_call(
        _reduce_body,
        grid=(grid,),
        in_specs=[pl.BlockSpec((blk, 128), lambda i: (i, 0)),
                  pl.BlockSpec((1, 1), lambda i: (0, 0))],
        out_specs=pl.BlockSpec((blk, 8), lambda i: (i, 0)),
        out_shape=jax.ShapeDtypeStruct((rows, 8), jnp.float32),
    )(x2, b2)


def kernel(z, edge_index, node_id, W1, b1, W2, b2):
    e = edge_index.shape[1]
    rows = edge_index[0]
    cols = edge_index[1]
    xn = lax.dynamic_slice_in_dim(z, node_id, 1, axis=0)
    P, Q = _build_tables(z, W1, b1.reshape(1, -1), xn)
    out16 = _sc_edge(P, Q, rows, cols, W2[:, 0])
    out8 = _reduce(out16.reshape(e * L // 128, 128), b2.reshape(1, 1))
    return out8.reshape(e, 1)


# speedup vs baseline: 11.3480x; 1.0002x over previous
import functools

import jax
import jax.numpy as jnp
from jax import lax
from jax.experimental import pallas as pl
from jax.experimental.pallas import tpu as pltpu
from jax.experimental.pallas import tpu_sc as plsc

NC = 2
NS = 16
NW = NC * NS
L = 16


def _tables_body(z_ref, w1_ref, b1_ref, xn_ref, p_ref, q_ref):
    c = jnp.dot(xn_ref[...], w1_ref[256:384, :],
                preferred_element_type=jnp.float32) + b1_ref[...]
    c = c * 0.5
    p_ref[...] = jnp.dot(z_ref[...], w1_ref[0:128, :],
                         preferred_element_type=jnp.float32) + c
    q_ref[...] = jnp.dot(z_ref[...], w1_ref[128:256, :],
                         preferred_element_type=jnp.float32) + c


def _build_tables(z, W1, b1, xn):
    n, _ = z.shape
    h = W1.shape[1]
    out = jax.ShapeDtypeStruct((n, h), jnp.float32)
    return pl.pallas_call(_tables_body, out_shape=(out, out))(z, W1, b1, xn)


def _sc_edge_body(p_hbm, q_hbm, rows_hbm, cols_hbm, w2_hbm, out_hbm,
                  ridx_v, cidx_v, qrows0, qrows1, prows0, prows1,
                  out0, out1, w2_v, qs0, qs1, ps0, ps1, os0, os1,
                  *, edges_per_w, blk, hidden):
    wid = lax.axis_index("c") * NS + lax.axis_index("s")
    base = wid * edges_per_w
    nstep = edges_per_w // blk
    nchunk = hidden // L
    pltpu.sync_copy(w2_hbm, w2_v)
    pltpu.sync_copy(rows_hbm.at[pl.ds(wid * nstep, nstep), :], ridx_v)
    pltpu.sync_copy(cols_hbm.at[pl.ds(wid * nstep, nstep), :], cidx_v)

    qrows = (qrows0, qrows1)
    prows = (prows0, prows1)
    outs = (out0, out1)
    qsems = (qs0, qs1)
    psems = (ps0, ps1)
    osems = (os0, os1)

    def gather_descs(i, b):
        dq = pltpu.make_async_copy(
            q_hbm.at[ridx_v.at[i]], qrows[b], qsems[b])
        dp = pltpu.make_async_copy(
            p_hbm.at[cidx_v.at[i]], prows[b], psems[b])
        return dq, dp

    def out_desc(i, b):
        return pltpu.make_async_copy(
            outs[b], out_hbm.at[pl.ds(base + i * blk, blk), :], osems[b])

    dq, dp = gather_descs(0, 0)
    dq.start()
    dp.start()

    def compute(b):
        pr, qr, ob = prows[b], qrows[b], outs[b]

        @functools.partial(plsc.parallel_loop, 0, blk, unroll=4)
        def _edge(e):
            s = None
            for ch in range(nchunk):
                sl = pl.ds(ch * L, L)
                t = jnp.maximum(pr[e, sl] + qr[e, sl], 0.0) * w2_v[sl]
                s = t if s is None else s + t
            ob[e, :] = s

    def process(j, b):
        @pl.when(j + 1 < nstep)
        def _():
            nq, np_ = gather_descs(j + 1, 1 - b)
            nq.start()
            np_.start()

        wq, wp = gather_descs(j, b)
        wq.wait()
        wp.wait()

        @pl.when(j >= 2)
        def _():
            out_desc(j - 2, b).wait()

        compute(b)
        out_desc(j, b).start()

    @pl.loop(0, nstep, step=2)
    def _(j):
        process(j, 0)
        process(j + 1, 1)

    out_desc(nstep - 2, 0).wait()
    out_desc(nstep - 1, 1).wait()


def _sc_edge(P, Q, rows, cols, w2):
    n, hidden = P.shape
    e = rows.shape[0]
    edges_per_w = e // NW
    blk = 200
    mesh = plsc.VectorSubcoreMesh(core_axis_name="c", subcore_axis_name="s")
    body = functools.partial(_sc_edge_body, edges_per_w=edges_per_w, blk=blk,
                             hidden=hidden)
    run = pl.kernel(
        body,
        out_type=jax.ShapeDtypeStruct((e, L), jnp.float32),
        mesh=mesh,
        compiler_params=pltpu.CompilerParams(use_tc_tiling_on_sc=False),
        scratch_types=[
            pltpu.VMEM((edges_per_w // blk, blk), jnp.int32),
            pltpu.VMEM((edges_per_w // blk, blk), jnp.int32),
            pltpu.VMEM((blk, hidden), jnp.float32),
            pltpu.VMEM((blk, hidden), jnp.float32),
            pltpu.VMEM((blk, hidden), jnp.float32),
            pltpu.VMEM((blk, hidden), jnp.float32),
            pltpu.VMEM((blk, L), jnp.float32),
            pltpu.VMEM((blk, L), jnp.float32),
            pltpu.VMEM((hidden,), jnp.float32),
            pltpu.SemaphoreType.DMA,
            pltpu.SemaphoreType.DMA,
            pltpu.SemaphoreType.DMA,
            pltpu.SemaphoreType.DMA,
            pltpu.SemaphoreType.DMA,
            pltpu.SemaphoreType.DMA,
        ],
    )
    return run(P, Q, rows.reshape(e // blk, blk), cols.reshape(e // blk, blk),
               w2)


def _reduce_body(x_ref, b2_ref, o_ref):
    sel = (lax.broadcasted_iota(jnp.int32, (128, 8), 0) // L
           == lax.broadcasted_iota(jnp.int32, (128, 8), 1))
    ones = sel.astype(jnp.float32)
    o_ref[...] = jnp.dot(x_ref[...], ones,
                         preferred_element_type=jnp.float32) + b2_ref[0, 0]


def _reduce(x2, b2):
    rows = x2.shape[0]
    grid = 10
    blk = rows // grid
    return pl.pallas_call(
        _reduce_body,
        grid=(grid,),
        in_specs=[pl.BlockSpec((blk, 128), lambda i: (i, 0)),
                  pl.BlockSpec((1, 1), lambda i: (0, 0))],
        out_specs=pl.BlockSpec((blk, 8), lambda i: (i, 0)),
        out_shape=jax.ShapeDtypeStruct((rows, 8), jnp.float32),
    )(x2, b2)


def kernel(z, edge_index, node_id, W1, b1, W2, b2):
    e = edge_index.shape[1]
    rows = edge_index[0]
    cols = edge_index[1]
    xn = lax.dynamic_slice_in_dim(z, node_id, 1, axis=0)
    P, Q = _build_tables(z, W1, b1.reshape(1, -1), xn)
    out16 = _sc_edge(P, Q, rows, cols, W2[:, 0])
    out8 = _reduce(out16.reshape(e * L // 128, 128), b2.reshape(1, 1))
    return out8.reshape(e, 1)
